# baseline (speedup 1.0000x reference)
import jax
import jax.numpy as jnp
from jax import lax
from jax.experimental import pallas as pl

B, S, H = 4, 4096, 2048
K = S // 2
TS = 512
NJ = S // TS

def _select_mask(row, k):
    i32_min = jnp.int32(-(2 ** 31))
    bits = lax.bitcast_convert_type(row, jnp.int32)
    ikey = jnp.where(bits < 0,
                     jnp.bitwise_xor(jnp.bitwise_not(bits), i32_min),
                     bits)
    cnt_nonneg = jnp.sum((ikey >= 0).astype(jnp.int32))
    base0 = jnp.where(cnt_nonneg >= k, jnp.int32(0), i32_min)

    def sbody(i, base):
        cand = base + lax.shift_left(jnp.int32(1), 30 - i)
        cnt = jnp.sum((ikey >= cand).astype(jnp.int32))
        return jnp.where(cnt >= k, cand, base)

    thr = lax.fori_loop(0, 31, sbody, base0)
    gt = ikey > thr
    eq = ikey == thr
    r = k - jnp.sum(gt.astype(jnp.int32))
    idx = (lax.broadcasted_iota(jnp.int32, (NJ, TS), 0) * TS
           + lax.broadcasted_iota(jnp.int32, (NJ, TS), 1))

    def tbody(i, m):
        trial = m + lax.shift_left(jnp.int32(1), 12 - i)
        cnt = jnp.sum((eq & (idx < trial)).astype(jnp.int32))
        return jnp.where(cnt <= r, trial, m)

    m = lax.fori_loop(0, 13, tbody, jnp.int32(0))
    return (gt | (eq & (idx < m))).astype(jnp.float32)


def _body(x_ref, w_ref, logits_ref, mask_ref):
    j = pl.program_id(1)
    xt = x_ref[0].astype(jnp.bfloat16)
    lt = lax.dot_general(xt, w_ref[...].astype(jnp.bfloat16),
                         dimension_numbers=(((1,), (0,)), ((), ())),
                         preferred_element_type=jnp.float32)
    logits_ref[0, j] = lt[:, 0]

    @pl.when(j == NJ - 1)
    def _():
        mask_ref[0] = _select_mask(logits_ref[0], K)


def kernel(x, w):
    w2 = w.reshape(H, 1)
    logits3, mask3 = pl.pallas_call(
        _body,
        grid=(B, NJ),
        in_specs=[
            pl.BlockSpec((1, TS, H), lambda b, j: (b, j, 0)),
            pl.BlockSpec((H, 1), lambda b, j: (0, 0)),
        ],
        out_specs=[
            pl.BlockSpec((1, NJ, TS), lambda b, j: (b, 0, 0)),
            pl.BlockSpec((1, NJ, TS), lambda b, j: (b, 0, 0)),
        ],
        out_shape=[
            jax.ShapeDtypeStruct((B, NJ, TS), jnp.float32),
            jax.ShapeDtypeStruct((B, NJ, TS), jnp.float32),
        ],
    )(x, w2)
    logits = logits3.reshape(B, S)
    mask = mask3.reshape(B, S)
    return (mask[..., None], mask, logits)
